# baseline (speedup 1.0000x reference)
import functools

import jax
import jax.numpy as jnp
from jax import lax
from jax.experimental import pallas as pl
from jax.experimental.pallas import tpu as pltpu
from jax.experimental.pallas import tpu_sc as plsc

NUM_NODES = 1000000
DIM = 64
BATCH = 16384

NC = 2
NS = 16
NW = NC * NS
BPW = BATCH // NW
GCHUNK = 128
NCHUNK = BPW // GCHUNK


def _sc_body(u_hbm, v_hbm, emb_hbm, ctx_hbm, out_hbm,
             idx_u, idx_v, erows, crows, out_v, sem_e, sem_c):
    wid = lax.axis_index("s") * NC + lax.axis_index("c")
    pltpu.sync_copy(u_hbm.at[wid], idx_u)
    pltpu.sync_copy(v_hbm.at[wid], idx_v)
    cps = []
    for j in range(NCHUNK):
        dst = erows.at[pl.ds(j * GCHUNK, GCHUNK)]
        cps.append(pltpu.async_copy(emb_hbm.at[idx_u.at[j]], dst, sem_e))
        dst = crows.at[pl.ds(j * GCHUNK, GCHUNK)]
        cps.append(pltpu.async_copy(ctx_hbm.at[idx_v.at[j]], dst, sem_c))
    for cp in cps:
        cp.wait()

    iota16 = lax.iota(jnp.int32, 16)
    perms = [iota16 ^ (1 << b) for b in range(4)]

    dnums = lax.GatherDimensionNumbers(
        offset_dims=(), collapsed_slice_dims=(0,), start_index_map=(0,))

    def permute(x, pm):
        return lax.gather(
            x, pm[:, None], dimension_numbers=dnums, slice_sizes=(1,),
            mode=lax.GatherScatterMode.PROMISE_IN_BOUNDS)

    def lanesum(p):
        for pm in perms:
            p = p + permute(p, pm)
        return p

    def group(g, carry):
        acc = jnp.zeros((16,), jnp.float32)
        for k in range(16):
            i = g * 16 + k
            p = erows[i, pl.ds(0, 16)] * crows[i, pl.ds(0, 16)]
            p += erows[i, pl.ds(16, 16)] * crows[i, pl.ds(16, 16)]
            p += erows[i, pl.ds(32, 16)] * crows[i, pl.ds(32, 16)]
            p += erows[i, pl.ds(48, 16)] * crows[i, pl.ds(48, 16)]
            acc = jnp.where(iota16 == k, lanesum(p), acc)
        out_v[pl.ds(g * 16, 16)] = acc
        return carry

    lax.fori_loop(0, BPW // 16, group, 0)
    pltpu.sync_copy(out_v, out_hbm.at[pl.ds(wid * BPW, BPW)])


_sc_dot = pl.kernel(
    _sc_body,
    out_type=jax.ShapeDtypeStruct((BATCH,), jnp.float32),
    mesh=plsc.VectorSubcoreMesh(core_axis_name="c", subcore_axis_name="s"),
    compiler_params=pltpu.CompilerParams(use_tc_tiling_on_sc=False),
    scratch_types=[
        pltpu.VMEM((NCHUNK, GCHUNK), jnp.int32),
        pltpu.VMEM((NCHUNK, GCHUNK), jnp.int32),
        pltpu.VMEM((BPW, DIM), jnp.float32),
        pltpu.VMEM((BPW, DIM), jnp.float32),
        pltpu.VMEM((BPW,), jnp.float32),
        pltpu.SemaphoreType.DMA,
        pltpu.SemaphoreType.DMA,
    ],
)


def _loss_body(p_ref, s_ref, o_ref):
    x = s_ref[...] * p_ref[...]
    ls = jnp.minimum(x, 0.0) - jnp.log1p(jnp.exp(-jnp.abs(x)))
    o_ref[0, 0] = -jnp.sum(ls)


_loss = pl.pallas_call(
    _loss_body,
    out_shape=jax.ShapeDtypeStruct((1, 1), jnp.float32),
    out_specs=pl.BlockSpec(memory_space=pltpu.SMEM),
)


def kernel(u, v, sign, emb_table, ctx_table):
    u3 = u.astype(jnp.int32).reshape(NW, NCHUNK, GCHUNK)
    v3 = v.astype(jnp.int32).reshape(NW, NCHUNK, GCHUNK)
    prod = _sc_dot(u3, v3, emb_table, ctx_table)
    loss = _loss(prod.reshape(128, 128), sign.reshape(128, 128))
    return loss.reshape(())


# speedup vs baseline: 1.5664x; 1.5664x over previous
import jax
import jax.numpy as jnp
from jax import lax
from jax.experimental import pallas as pl
from jax.experimental.pallas import tpu as pltpu
from jax.experimental.pallas import tpu_sc as plsc

NUM_NODES = 1000000
DIM = 64
BATCH = 16384

NC = 2
NS = 16
NW = NC * NS
BPW = BATCH // NW
CH = 128
NCH = BPW // CH
NSLOT = 2


def _sc_body(u_hbm, v_hbm, emb_hbm, ctx_hbm, out_hbm,
             idx_u, idx_v, erows, crows, out_v, sem_e, sem_c):
    wid = lax.axis_index("s") * NC + lax.axis_index("c")
    pltpu.sync_copy(u_hbm.at[wid], idx_u)
    pltpu.sync_copy(v_hbm.at[wid], idx_v)

    def fire(c, s):
        def body(g, carry):
            vu = idx_u[pl.ds(c * CH + g * 16, 16)]
            vv = idx_v[pl.ds(c * CH + g * 16, 16)]
            for k in range(16):
                r = g * 16 + k
                pltpu.async_copy(
                    emb_hbm.at[vu[k]], erows.at[s, r], sem_e.at[s])
                pltpu.async_copy(
                    ctx_hbm.at[vv[k]], crows.at[s, r], sem_c.at[s])
            return carry

        lax.fori_loop(0, CH // 16, body, 0)

    def drain(s):
        pltpu.make_async_copy(
            emb_hbm.at[pl.ds(0, CH)], erows.at[s], sem_e.at[s]).wait()
        pltpu.make_async_copy(
            ctx_hbm.at[pl.ds(0, CH)], crows.at[s], sem_c.at[s]).wait()

    iota16 = lax.iota(jnp.int32, 16)
    perms = [iota16 ^ (1 << b) for b in range(4)]
    dnums = lax.GatherDimensionNumbers(
        offset_dims=(), collapsed_slice_dims=(0,), start_index_map=(0,))

    def permute(x, pm):
        return lax.gather(
            x, pm[:, None], dimension_numbers=dnums, slice_sizes=(1,),
            mode=lax.GatherScatterMode.PROMISE_IN_BOUNDS)

    def lanesum(p):
        for pm in perms:
            p = p + permute(p, pm)
        return p

    def compute(c, s):
        def body(g, carry):
            acc = jnp.zeros((16,), jnp.float32)
            for k in range(16):
                r = g * 16 + k
                p = erows[s, r, pl.ds(0, 16)] * crows[s, r, pl.ds(0, 16)]
                p += erows[s, r, pl.ds(16, 16)] * crows[s, r, pl.ds(16, 16)]
                p += erows[s, r, pl.ds(32, 16)] * crows[s, r, pl.ds(32, 16)]
                p += erows[s, r, pl.ds(48, 16)] * crows[s, r, pl.ds(48, 16)]
                acc = jnp.where(iota16 == k, lanesum(p), acc)
            out_v[pl.ds(c * CH + g * 16, 16)] = acc
            return carry

        lax.fori_loop(0, CH // 16, body, 0)

    for c in range(NSLOT):
        fire(c, c)
    for c in range(NCH):
        s = c % NSLOT
        drain(s)
        compute(c, s)
        if c + NSLOT < NCH:
            fire(c + NSLOT, s)

    pltpu.sync_copy(out_v, out_hbm.at[pl.ds(wid * BPW, BPW)])


_sc_dot = pl.kernel(
    _sc_body,
    out_type=jax.ShapeDtypeStruct((BATCH,), jnp.float32),
    mesh=plsc.VectorSubcoreMesh(core_axis_name="c", subcore_axis_name="s"),
    scratch_types=[
        pltpu.VMEM((BPW,), jnp.int32),
        pltpu.VMEM((BPW,), jnp.int32),
        pltpu.VMEM((NSLOT, CH, DIM), jnp.float32),
        pltpu.VMEM((NSLOT, CH, DIM), jnp.float32),
        pltpu.VMEM((BPW,), jnp.float32),
        pltpu.SemaphoreType.DMA((NSLOT,)),
        pltpu.SemaphoreType.DMA((NSLOT,)),
    ],
)


def _loss_body(p_ref, s_ref, o_ref):
    x = s_ref[...] * p_ref[...]
    ls = jnp.minimum(x, 0.0) - jnp.log1p(jnp.exp(-jnp.abs(x)))
    o_ref[0, 0] = -jnp.sum(ls)


_loss = pl.pallas_call(
    _loss_body,
    out_shape=jax.ShapeDtypeStruct((1, 1), jnp.float32),
    out_specs=pl.BlockSpec(memory_space=pltpu.SMEM),
)


def kernel(u, v, sign, emb_table, ctx_table):
    u2 = u.astype(jnp.int32).reshape(NW, BPW)
    v2 = v.astype(jnp.int32).reshape(NW, BPW)
    prod = _sc_dot(u2, v2, emb_table, ctx_table)
    loss = _loss(prod.reshape(128, 128), sign.reshape(128, 128))
    return loss.reshape(())


# speedup vs baseline: 1.5688x; 1.0016x over previous
import jax
import jax.numpy as jnp
from jax import lax
from jax.experimental import pallas as pl
from jax.experimental.pallas import tpu as pltpu
from jax.experimental.pallas import tpu_sc as plsc

NUM_NODES = 1000000
DIM = 64
BATCH = 16384

NC = 2
NS = 16
NW = NC * NS
BPW = BATCH // NW
CH = 128
NCH = BPW // CH
NSLOT = 2


def _sc_body(u_hbm, v_hbm, emb_hbm, ctx_hbm, out_hbm,
             idx_u, idx_v, erows, crows, out_v, sem_e, sem_c):
    wid = lax.axis_index("s") * NC + lax.axis_index("c")
    pltpu.sync_copy(u_hbm.at[wid], idx_u)
    pltpu.sync_copy(v_hbm.at[wid], idx_v)

    def fire(c, s):
        def body(g, carry):
            vu = idx_u[pl.ds(c * CH + g * 16, 16)]
            vv = idx_v[pl.ds(c * CH + g * 16, 16)]
            for k in range(16):
                r = g * 16 + k
                pltpu.async_copy(
                    emb_hbm.at[vu[k]], erows.at[s, r], sem_e.at[s])
                pltpu.async_copy(
                    ctx_hbm.at[vv[k]], crows.at[s, r], sem_c.at[s])
            return carry

        lax.fori_loop(0, CH // 16, body, 0)

    def drain(s):
        pltpu.make_async_copy(
            emb_hbm.at[pl.ds(0, CH)], erows.at[s], sem_e.at[s]).wait()
        pltpu.make_async_copy(
            ctx_hbm.at[pl.ds(0, CH)], crows.at[s], sem_c.at[s]).wait()

    iota16 = lax.iota(jnp.int32, 16)
    perms = [iota16 ^ (1 << b) for b in range(4)]
    dnums = lax.GatherDimensionNumbers(
        offset_dims=(), collapsed_slice_dims=(0,), start_index_map=(0,))

    def permute(x, pm):
        return lax.gather(
            x, pm[:, None], dimension_numbers=dnums, slice_sizes=(1,),
            mode=lax.GatherScatterMode.PROMISE_IN_BOUNDS)

    def lanesum(p):
        for pm in perms:
            p = p + permute(p, pm)
        return p

    def compute(c, s):
        def body(g, carry):
            acc = jnp.zeros((16,), jnp.float32)
            for k in range(16):
                r = g * 16 + k
                p = erows[s, r, pl.ds(0, 16)] * crows[s, r, pl.ds(0, 16)]
                p += erows[s, r, pl.ds(16, 16)] * crows[s, r, pl.ds(16, 16)]
                p += erows[s, r, pl.ds(32, 16)] * crows[s, r, pl.ds(32, 16)]
                p += erows[s, r, pl.ds(48, 16)] * crows[s, r, pl.ds(48, 16)]
                acc = jnp.where(iota16 == k, lanesum(p), acc)
            out_v[pl.ds(c * CH + g * 16, 16)] = acc
            return carry

        lax.fori_loop(0, CH // 16, body, 0)

    for c in range(NSLOT):
        fire(c, c)
    for c in range(NCH):
        s = c % NSLOT
        drain(s)
        compute(c, s)
        if c + NSLOT < NCH:
            fire(c + NSLOT, s)

    pltpu.sync_copy(out_v, out_hbm.at[pl.ds(wid * BPW, BPW)])


_sc_dot = pl.kernel(
    _sc_body,
    out_type=jax.ShapeDtypeStruct((BATCH,), jnp.float32),
    mesh=plsc.VectorSubcoreMesh(core_axis_name="c", subcore_axis_name="s"),
    compiler_params=pltpu.CompilerParams(skip_device_barrier=True),
    scratch_types=[
        pltpu.VMEM((BPW,), jnp.int32),
        pltpu.VMEM((BPW,), jnp.int32),
        pltpu.VMEM((NSLOT, CH, DIM), jnp.float32),
        pltpu.VMEM((NSLOT, CH, DIM), jnp.float32),
        pltpu.VMEM((BPW,), jnp.float32),
        pltpu.SemaphoreType.DMA((NSLOT,)),
        pltpu.SemaphoreType.DMA((NSLOT,)),
    ],
)


def _loss_body(p_ref, s_ref, o_ref):
    x = s_ref[...] * p_ref[...]
    ls = jnp.minimum(x, 0.0) - jnp.log1p(jnp.exp(-jnp.abs(x)))
    o_ref[0, 0] = -jnp.sum(ls)


_loss = pl.pallas_call(
    _loss_body,
    out_shape=jax.ShapeDtypeStruct((1, 1), jnp.float32),
    out_specs=pl.BlockSpec(memory_space=pltpu.SMEM),
)


def kernel(u, v, sign, emb_table, ctx_table):
    u2 = u.astype(jnp.int32).reshape(NW, BPW)
    v2 = v.astype(jnp.int32).reshape(NW, BPW)
    prod = _sc_dot(u2, v2, emb_table, ctx_table)
    loss = _loss(prod.reshape(128, 128), sign.reshape(128, 128))
    return loss.reshape(())
